# Initial kernel scaffold; baseline (speedup 1.0000x reference)
import jax
import jax.numpy as jnp
from jax.experimental import pallas as pl


def kernel(raw_obs_batch, positions_batch, W1, b1, W2, b2, cW1, cb1, cW2, cb2, cW3, cb3):
    raise NotImplementedError("write your pallas kernel here")


# speedup vs baseline: 81.8347x; 81.8347x over previous
import jax
import jax.numpy as jnp
from jax.experimental import pallas as pl

DIST = 0.32

_HI = jax.lax.Precision.HIGHEST


def _body(pos_col_ref, pos_row_ref, x_ref,
          W1_ref, b1_ref, W2_ref, b2_ref,
          cW1_ref, cb1_ref, cW2_ref, cb2_ref, cW3_ref, cb3_ref,
          out_ref):
    pc = pos_col_ref[0]
    pr = pos_row_ref[0]
    px_c = pc[:, 0:1]
    py_c = pc[:, 1:2]
    px_r = pr[0:1, :]
    py_r = pr[1:2, :]
    dx = px_c - px_r
    dy = py_c - py_r
    dist = jnp.sqrt(dx * dx + dy * dy)
    mask = (dist <= DIST).astype(jnp.float32)
    degree = jnp.sum(mask, axis=1, keepdims=True)
    inv_deg = 1.0 / jnp.maximum(degree, 1.0)

    x = x_ref[0]
    agg = jnp.dot(mask, x, precision=_HI)
    h = jnp.maximum((agg * inv_deg) @ W1_ref[...] + b1_ref[0:1, :], 0.0)
    agg = jnp.dot(mask, h, precision=_HI)
    h = jnp.maximum((agg * inv_deg) @ W2_ref[...] + b2_ref[0:1, :], 0.0)
    v = jnp.tanh(h @ cW1_ref[...] + cb1_ref[0:1, :])
    v = jnp.tanh(v @ cW2_ref[...] + cb2_ref[0:1, :])
    out_ref[0] = v @ cW3_ref[...] + cb3_ref[0:1, :]


def kernel(raw_obs_batch, positions_batch, W1, b1, W2, b2,
           cW1, cb1, cW2, cb2, cW3, cb3):
    E, A, F = raw_obs_batch.shape
    H2 = cW2.shape[0]

    pos_col = jnp.pad(positions_batch, ((0, 0), (0, 0), (0, 6)))
    pos_row = jnp.pad(jnp.transpose(positions_batch, (0, 2, 1)),
                      ((0, 0), (0, 6), (0, 0)))
    cW3p = jnp.pad(cW3, ((0, 0), (0, 127)))
    cb3p = jnp.pad(cb3.reshape(1, 1), ((0, 0), (0, 127)))

    env = lambda e: (e, 0, 0)
    const2 = lambda e: (0, 0)

    out = pl.pallas_call(
        _body,
        grid=(E,),
        in_specs=[
            pl.BlockSpec((1, A, 8), env),
            pl.BlockSpec((1, 8, A), env),
            pl.BlockSpec((1, A, F), env),
            pl.BlockSpec(W1.shape, const2),
            pl.BlockSpec((1, F), const2),
            pl.BlockSpec(W2.shape, const2),
            pl.BlockSpec((1, F), const2),
            pl.BlockSpec(cW1.shape, const2),
            pl.BlockSpec((1, H2), const2),
            pl.BlockSpec(cW2.shape, const2),
            pl.BlockSpec((1, H2), const2),
            pl.BlockSpec((H2, 128), const2),
            pl.BlockSpec((1, 128), const2),
        ],
        out_specs=pl.BlockSpec((1, A, 128), env),
        out_shape=jax.ShapeDtypeStruct((E, A, 128), jnp.float32),
    )(pos_col, pos_row, raw_obs_batch, W1, b1.reshape(1, -1),
      W2, b2.reshape(1, -1), cW1, cb1.reshape(1, -1),
      cW2, cb2.reshape(1, -1), cW3p, cb3p)

    return out[:, :, 0:1].reshape(E * A, 1)


# speedup vs baseline: 188.7120x; 2.3060x over previous
import jax
import jax.numpy as jnp
from jax.experimental import pallas as pl

DIST = 0.32
_AP = 104
_B = 8

_HI = jax.lax.Precision.HIGHEST


def _body(pos_col_ref, pos_row_ref, x_ref,
          W1_ref, b1_ref, W2_ref, b2_ref,
          cW1_ref, cb1_ref, cW2_ref, cb2_ref, cW3_ref, cb3_ref,
          out_ref):
    masks = []
    inv_degs = []
    for b in range(_B):
        pc = pos_col_ref[b * _AP:(b + 1) * _AP, :]
        pr = pos_row_ref[b * 8:(b + 1) * 8, :]
        dx = pc[:, 0:1] - pr[0:1, :]
        dy = pc[:, 1:2] - pr[1:2, :]
        dist = jnp.sqrt(dx * dx + dy * dy)
        mask = (dist <= DIST).astype(jnp.float32)
        degree = jnp.sum(mask, axis=1, keepdims=True)
        masks.append(mask)
        inv_degs.append(1.0 / jnp.maximum(degree, 1.0))

    mean1 = [jnp.dot(masks[b], x_ref[b * _AP:(b + 1) * _AP, :],
                     precision=_HI) * inv_degs[b] for b in range(_B)]
    h = jnp.concatenate(mean1, axis=0)
    h = jnp.maximum(h @ W1_ref[...] + b1_ref[0:1, :], 0.0)
    mean2 = [jnp.dot(masks[b], h[b * _AP:(b + 1) * _AP, :],
                     precision=_HI) * inv_degs[b] for b in range(_B)]
    h = jnp.concatenate(mean2, axis=0)
    h = jnp.maximum(h @ W2_ref[...] + b2_ref[0:1, :], 0.0)
    v = jnp.tanh(h @ cW1_ref[...] + cb1_ref[0:1, :])
    v = jnp.tanh(v @ cW2_ref[...] + cb2_ref[0:1, :])
    out_ref[...] = v @ cW3_ref[...] + cb3_ref[0:1, :]


def kernel(raw_obs_batch, positions_batch, W1, b1, W2, b2,
           cW1, cb1, cW2, cb2, cW3, cb3):
    E, A, F = raw_obs_batch.shape
    H2 = cW2.shape[0]
    pad = _AP - A

    obs_p = jnp.pad(raw_obs_batch, ((0, 0), (0, pad), (0, 0)))
    pos_p = jnp.pad(positions_batch, ((0, 0), (0, pad), (0, 0)),
                    constant_values=1e6)
    pos_col = jnp.pad(pos_p, ((0, 0), (0, 0), (0, 6))).reshape(E * _AP, 8)
    pos_row = jnp.pad(jnp.transpose(pos_p, (0, 2, 1)),
                      ((0, 0), (0, 6), (0, 0))).reshape(E * 8, _AP)
    x_flat = obs_p.reshape(E * _AP, F)
    cW3p = jnp.pad(cW3, ((0, 0), (0, 127)))
    cb3p = jnp.pad(cb3.reshape(1, 1), ((0, 0), (0, 127)))

    row = lambda i: (i, 0)
    const = lambda i: (0, 0)

    out = pl.pallas_call(
        _body,
        grid=(E // _B,),
        in_specs=[
            pl.BlockSpec((_B * _AP, 8), row),
            pl.BlockSpec((_B * 8, _AP), row),
            pl.BlockSpec((_B * _AP, F), row),
            pl.BlockSpec(W1.shape, const),
            pl.BlockSpec((1, F), const),
            pl.BlockSpec(W2.shape, const),
            pl.BlockSpec((1, F), const),
            pl.BlockSpec(cW1.shape, const),
            pl.BlockSpec((1, H2), const),
            pl.BlockSpec(cW2.shape, const),
            pl.BlockSpec((1, H2), const),
            pl.BlockSpec((H2, 128), const),
            pl.BlockSpec((1, 128), const),
        ],
        out_specs=pl.BlockSpec((_B * _AP, 128), row),
        out_shape=jax.ShapeDtypeStruct((E * _AP, 128), jnp.float32),
    )(pos_col, pos_row, x_flat, W1, b1.reshape(1, -1),
      W2, b2.reshape(1, -1), cW1, cb1.reshape(1, -1),
      cW2, cb2.reshape(1, -1), cW3p, cb3p)

    return out.reshape(E, _AP, 128)[:, :A, 0:1].reshape(E * A, 1)


# speedup vs baseline: 286.9174x; 1.5204x over previous
import jax
import jax.numpy as jnp
from jax.experimental import pallas as pl

DIST = 0.32
_AP = 104
_B = 8


def _body(pos_col_ref, pos_row_ref, x_ref,
          W1_ref, b1_ref, W2_ref, b2_ref,
          cW1_ref, cb1_ref, cW2_ref, cb2_ref, cW3_ref, cb3_ref,
          out_ref):
    A = x_ref.shape[1]
    masks = []
    inv_degs = []
    for b in range(_B):
        pc = pos_col_ref[b]
        pr = pos_row_ref[b]
        dx = pc[:, 0:1] - pr[0:1, :]
        dy = pc[:, 1:2] - pr[1:2, :]
        dist = jnp.sqrt(dx * dx + dy * dy)
        mask = (dist <= DIST).astype(jnp.float32)
        degree = jnp.sum(mask, axis=1, keepdims=True)
        masks.append(mask)
        inv_degs.append(1.0 / jnp.maximum(degree, 1.0))

    mean1 = [jnp.dot(masks[b][:, :A], x_ref[b]) * inv_degs[b]
             for b in range(_B)]
    h = jnp.concatenate(mean1, axis=0)
    h = jnp.maximum(h @ W1_ref[...] + b1_ref[0:1, :], 0.0)
    mean2 = [jnp.dot(masks[b], h[b * _AP:(b + 1) * _AP, :]) * inv_degs[b]
             for b in range(_B)]
    h = jnp.concatenate(mean2, axis=0)
    h = jnp.maximum(h @ W2_ref[...] + b2_ref[0:1, :], 0.0)
    v = jnp.tanh(h @ cW1_ref[...] + cb1_ref[0:1, :])
    v = jnp.tanh(v @ cW2_ref[...] + cb2_ref[0:1, :])
    out_ref[...] = v @ cW3_ref[...] + cb3_ref[0:1, :]


def kernel(raw_obs_batch, positions_batch, W1, b1, W2, b2,
           cW1, cb1, cW2, cb2, cW3, cb3):
    E, A, F = raw_obs_batch.shape
    H2 = cW2.shape[0]
    pad = _AP - A

    pos_p = jnp.pad(positions_batch, ((0, 0), (0, pad), (0, 0)),
                    constant_values=1e6)
    pos_col = jnp.pad(pos_p, ((0, 0), (0, 0), (0, 6)))
    pos_row = jnp.pad(jnp.transpose(pos_p, (0, 2, 1)),
                      ((0, 0), (0, 6), (0, 0)))
    cW3p = jnp.pad(cW3, ((0, 0), (0, 7)))
    cb3p = jnp.pad(cb3.reshape(1, 1), ((0, 0), (0, 7)))

    env3 = lambda i: (i, 0, 0)
    row = lambda i: (i, 0)
    const = lambda i: (0, 0)

    out = pl.pallas_call(
        _body,
        grid=(E // _B,),
        in_specs=[
            pl.BlockSpec((_B, _AP, 8), env3),
            pl.BlockSpec((_B, 8, _AP), env3),
            pl.BlockSpec((_B, A, F), env3),
            pl.BlockSpec(W1.shape, const),
            pl.BlockSpec((1, F), const),
            pl.BlockSpec(W2.shape, const),
            pl.BlockSpec((1, F), const),
            pl.BlockSpec(cW1.shape, const),
            pl.BlockSpec((1, H2), const),
            pl.BlockSpec(cW2.shape, const),
            pl.BlockSpec((1, H2), const),
            pl.BlockSpec((H2, 8), const),
            pl.BlockSpec((1, 8), const),
        ],
        out_specs=pl.BlockSpec((_B * _AP, 8), row),
        out_shape=jax.ShapeDtypeStruct((E * _AP, 8), jnp.float32),
    )(pos_col, pos_row, raw_obs_batch, W1, b1.reshape(1, -1),
      W2, b2.reshape(1, -1), cW1, cb1.reshape(1, -1),
      cW2, cb2.reshape(1, -1), cW3p, cb3p)

    return out.reshape(E, _AP, 8)[:, :A, 0:1].reshape(E * A, 1)
